# baseline (speedup 1.0000x reference)
import functools

import jax
import jax.numpy as jnp
from jax import lax
from jax.experimental import pallas as pl
from jax.experimental.pallas import tpu as pltpu
from jax.experimental.pallas import tpu_sc as plsc

N = 10000
E = 160000
D = 256
Q = 64
CH = 128
NC = 2
NS = 16
E_PAD = 163840
NCHUNK = E_PAD // CH
TCH = NCHUNK // NS
U = 4
NU = TCH // U
NP = 10240
PT = NP // NS

_mesh = plsc.VectorSubcoreMesh(core_axis_name="c", subcore_axis_name="s")


@functools.partial(
    pl.kernel,
    out_type=jax.ShapeDtypeStruct((NC, NP, Q), jnp.float32),
    mesh=_mesh,
    scratch_types=[
        pltpu.VMEM((NU, U * CH), jnp.int32),
        pltpu.VMEM((NU, U * CH), jnp.int32),
        pltpu.VMEM((2, U * CH, Q), jnp.float32),
        pltpu.VMEM_SHARED((NP, Q), jnp.float32),
        pltpu.SemaphoreType.DMA,
        pltpu.SemaphoreType.DMA,
    ],
    compiler_params=pltpu.CompilerParams(use_tc_tiling_on_sc=False),
)
def _sc_round(nfq_hbm, src2_hbm, dst_hbm, out_sum,
              isrc, idst, rows, acc_sh, gsem, ssem):
    c = lax.axis_index("c")
    s = lax.axis_index("s")
    zeros16 = jnp.zeros((16,), jnp.float32)
    base_r = s * PT

    pltpu.sync_copy(src2_hbm.at[c, s], isrc)
    pltpu.sync_copy(dst_hbm.at[s], idst)

    def zrow(i, _):
        for j in range(Q // 16):
            rows[0, i, pl.ds(j * 16, 16)] = zeros16
        return 0
    lax.fori_loop(0, CH, zrow, 0)
    for t in range(PT // CH):
        pltpu.sync_copy(rows.at[0, pl.ds(0, CH)],
                        acc_sh.at[pl.ds(base_r + t * CH, CH)])
    plsc.subcore_barrier()

    def pair(p, _):
        u0 = 2 * p
        g0 = pltpu.async_copy(nfq_hbm.at[isrc.at[u0]], rows.at[0], gsem)
        g1 = pltpu.async_copy(nfq_hbm.at[isrc.at[u0 + 1]], rows.at[1], gsem)
        g0.wait()
        s0 = pltpu.async_copy(rows.at[0], acc_sh.at[idst.at[u0]],
                              ssem, add=True)
        g1.wait()
        s1 = pltpu.async_copy(rows.at[1], acc_sh.at[idst.at[u0 + 1]],
                              ssem, add=True)
        s0.wait()
        s1.wait()
        return 0
    lax.fori_loop(0, NU // 2, pair, 0)
    plsc.subcore_barrier()

    for t in range(PT // CH):
        r0 = base_r + t * CH
        pltpu.sync_copy(acc_sh.at[pl.ds(r0, CH)], rows.at[0, pl.ds(0, CH)])
        pltpu.sync_copy(rows.at[0, pl.ds(0, CH)],
                        out_sum.at[c, pl.ds(r0, CH)])


TCHD = NCHUNK // (NC * NS)
RD = 8
GRPD = TCHD // RD


@functools.partial(
    pl.kernel,
    out_type=jax.ShapeDtypeStruct((NC, NP, 16), jnp.float32),
    mesh=_mesh,
    scratch_types=[
        pltpu.VMEM((TCHD, CH), jnp.int32),
        pltpu.VMEM((CH, 16), jnp.float32),
        pltpu.VMEM_SHARED((NP, 16), jnp.float32),
        pltpu.SemaphoreType.DMA,
    ],
    compiler_params=pltpu.CompilerParams(use_tc_tiling_on_sc=False),
)
def _sc_degree(dst_hbm, out_deg, idst, ones_b, deg_sh, dsem):
    c = lax.axis_index("c")
    s = lax.axis_index("s")
    zeros16 = jnp.zeros((16,), jnp.float32)
    ones16 = jnp.ones((16,), jnp.float32)
    base_r = s * PT

    pltpu.sync_copy(dst_hbm.at[pl.ds((c * NS + s) * TCHD, TCHD)], idst)

    def zdeg(i, _):
        ones_b[i] = zeros16
        return 0
    lax.fori_loop(0, CH, zdeg, 0)
    for t in range(PT // CH):
        pltpu.sync_copy(ones_b, deg_sh.at[pl.ds(base_r + t * CH, CH)])

    def orow(i, _):
        ones_b[i] = ones16
        return 0
    lax.fori_loop(0, CH, orow, 0)
    plsc.subcore_barrier()

    def group(g, _):
        cb = g * RD
        dds = [pltpu.async_copy(ones_b, deg_sh.at[idst.at[cb + k]],
                                dsem, add=True) for k in range(RD)]
        for d in dds:
            d.wait()
        return 0
    lax.fori_loop(0, GRPD, group, 0)
    plsc.subcore_barrier()

    for t in range(PT // CH):
        r0 = base_r + t * CH
        pltpu.sync_copy(deg_sh.at[pl.ds(r0, CH)], ones_b)
        pltpu.sync_copy(ones_b, out_deg.at[c, pl.ds(r0, CH)])


_BLK = 1000


def _tc_body(nf, sm, dg, w1, b1, w2, b2, eps, out):
    deg = jnp.maximum(dg[...], 1.0)
    mean = sm[...] / deg
    rst = (1.0 + eps[0, 0]) * nf[...] + mean
    h = jnp.maximum(
        jnp.dot(rst, w1[...], preferred_element_type=jnp.float32) + b1[...], 0.0)
    out[...] = jnp.dot(h, w2[...], preferred_element_type=jnp.float32) + b2[...]


def _tc_apply(nf, sm, deg, W1, b1, W2, b2, eps):
    return pl.pallas_call(
        _tc_body,
        grid=(N // _BLK,),
        in_specs=[
            pl.BlockSpec((_BLK, D), lambda i: (i, 0)),
            pl.BlockSpec((_BLK, D), lambda i: (i, 0)),
            pl.BlockSpec((_BLK, 1), lambda i: (i, 0)),
            pl.BlockSpec((D, D), lambda i: (0, 0)),
            pl.BlockSpec((1, D), lambda i: (0, 0)),
            pl.BlockSpec((D, D), lambda i: (0, 0)),
            pl.BlockSpec((1, D), lambda i: (0, 0)),
            pl.BlockSpec((1, 1), lambda i: (0, 0)),
        ],
        out_specs=pl.BlockSpec((_BLK, D), lambda i: (i, 0)),
        out_shape=jax.ShapeDtypeStruct((N, D), jnp.float32),
    )(nf, sm, deg, W1, b1, W2, b2, eps)


def kernel(node_feat, coord_feat, edge_feat, edge_index, W1, b1, W2, b2, eps):
    src = edge_index[0].astype(jnp.int32)
    dst = edge_index[1].astype(jnp.int32)
    pad = E_PAD - E
    src_p = jnp.concatenate([src, jnp.zeros((pad,), jnp.int32)])
    dst_p = jnp.concatenate([dst, jnp.full((pad,), N, jnp.int32)])
    srcq = (src_p[None, :]
            + (jnp.arange(4, dtype=jnp.int32) * N)[:, None])
    src01 = srcq[0:2].reshape(NC, NS, NU, U * CH)
    src23 = srcq[2:4].reshape(NC, NS, NU, U * CH)
    dst_u = dst_p.reshape(NS, NU, U * CH)
    dst_c = dst_p.reshape(NCHUNK, CH)
    nfq = node_feat.reshape(N, 4, Q).transpose(1, 0, 2).reshape(4 * N, Q)

    sum01 = _sc_round(nfq, src01, dst_u)
    sum23 = _sc_round(nfq, src23, dst_u)
    deg2 = _sc_degree(dst_c)
    summed = jnp.concatenate(
        [sum01[0, :N], sum01[1, :N], sum23[0, :N], sum23[1, :N]], axis=1)
    deg = (deg2[0, :N, 0] + deg2[1, :N, 0])[:, None]

    hx = _tc_apply(node_feat, summed, deg, W1, jnp.reshape(b1, (1, D)),
                   W2, jnp.reshape(b2, (1, D)),
                   jnp.reshape(eps, (1, 1)).astype(jnp.float32))
    return (hx, coord_feat, edge_feat)


# speedup vs baseline: 1.0953x; 1.0953x over previous
import functools

import jax
import jax.numpy as jnp
from jax import lax
from jax.experimental import pallas as pl
from jax.experimental.pallas import tpu as pltpu
from jax.experimental.pallas import tpu_sc as plsc

N = 10000
E = 160000
D = 256
H = 128
CH = 128
NC = 2
NS = 16
E_PAD = 163840
NCHUNK = E_PAD // CH
TCH = NCHUNK // NS
HCH = TCH // 2
NP = 10240
PT = NP // NS

_mesh = plsc.VectorSubcoreMesh(core_axis_name="c", subcore_axis_name="s")


@functools.partial(
    pl.kernel,
    out_type=jax.ShapeDtypeStruct((NC, NP, H), jnp.float32),
    mesh=_mesh,
    scratch_types=[
        pltpu.VMEM((HCH, CH), jnp.int32),
        pltpu.VMEM((HCH, CH), jnp.int32),
        pltpu.VMEM((2, CH, H), jnp.float32),
        pltpu.VMEM_SHARED((NP, H), jnp.float32),
        pltpu.SemaphoreType.DMA,
        pltpu.SemaphoreType.DMA,
    ],
    compiler_params=pltpu.CompilerParams(use_tc_tiling_on_sc=False),
)
def _sc_halves(nfh_hbm, src2_hbm, dst_hbm, out_sum,
               isrc, idst, rows, acc_sh, gsem, ssem):
    c = lax.axis_index("c")
    s = lax.axis_index("s")
    zeros16 = jnp.zeros((16,), jnp.float32)
    base_r = s * PT

    def zrow(i, _):
        for j in range(H // 16):
            rows[0, i, pl.ds(j * 16, 16)] = zeros16
        return 0
    lax.fori_loop(0, CH, zrow, 0)
    for t in range(PT // CH):
        pltpu.sync_copy(rows.at[0], acc_sh.at[pl.ds(base_r + t * CH, CH)])
    plsc.subcore_barrier()

    for h in range(2):
        pltpu.sync_copy(src2_hbm.at[c, s, pl.ds(h * HCH, HCH)], isrc)
        pltpu.sync_copy(dst_hbm.at[s, pl.ds(h * HCH, HCH)], idst)

        def pair(p, _):
            u0 = 2 * p
            g0 = pltpu.async_copy(nfh_hbm.at[isrc.at[u0]], rows.at[0], gsem)
            g1 = pltpu.async_copy(nfh_hbm.at[isrc.at[u0 + 1]], rows.at[1],
                                  gsem)
            g0.wait()
            s0 = pltpu.async_copy(rows.at[0], acc_sh.at[idst.at[u0]],
                                  ssem, add=True)
            g1.wait()
            s1 = pltpu.async_copy(rows.at[1], acc_sh.at[idst.at[u0 + 1]],
                                  ssem, add=True)
            s0.wait()
            s1.wait()
            return 0
        lax.fori_loop(0, HCH // 2, pair, 0)
    plsc.subcore_barrier()

    for t in range(PT // CH):
        r0 = base_r + t * CH
        pltpu.sync_copy(acc_sh.at[pl.ds(r0, CH)], rows.at[0])
        pltpu.sync_copy(rows.at[0], out_sum.at[c, pl.ds(r0, CH)])


TCHD = NCHUNK // (NC * NS)
RD = 8
GRPD = TCHD // RD


@functools.partial(
    pl.kernel,
    out_type=jax.ShapeDtypeStruct((NC, NP, 16), jnp.float32),
    mesh=_mesh,
    scratch_types=[
        pltpu.VMEM((TCHD, CH), jnp.int32),
        pltpu.VMEM((CH, 16), jnp.float32),
        pltpu.VMEM_SHARED((NP, 16), jnp.float32),
        pltpu.SemaphoreType.DMA,
    ],
    compiler_params=pltpu.CompilerParams(use_tc_tiling_on_sc=False),
)
def _sc_degree(dst_hbm, out_deg, idst, ones_b, deg_sh, dsem):
    c = lax.axis_index("c")
    s = lax.axis_index("s")
    zeros16 = jnp.zeros((16,), jnp.float32)
    ones16 = jnp.ones((16,), jnp.float32)
    base_r = s * PT

    pltpu.sync_copy(dst_hbm.at[pl.ds((c * NS + s) * TCHD, TCHD)], idst)

    def zdeg(i, _):
        ones_b[i] = zeros16
        return 0
    lax.fori_loop(0, CH, zdeg, 0)
    for t in range(PT // CH):
        pltpu.sync_copy(ones_b, deg_sh.at[pl.ds(base_r + t * CH, CH)])

    def orow(i, _):
        ones_b[i] = ones16
        return 0
    lax.fori_loop(0, CH, orow, 0)
    plsc.subcore_barrier()

    def group(g, _):
        cb = g * RD
        dds = [pltpu.async_copy(ones_b, deg_sh.at[idst.at[cb + k]],
                                dsem, add=True) for k in range(RD)]
        for d in dds:
            d.wait()
        return 0
    lax.fori_loop(0, GRPD, group, 0)
    plsc.subcore_barrier()

    for t in range(PT // CH):
        r0 = base_r + t * CH
        pltpu.sync_copy(deg_sh.at[pl.ds(r0, CH)], ones_b)
        pltpu.sync_copy(ones_b, out_deg.at[c, pl.ds(r0, CH)])


_BLK = 1000


def _tc_body(nf, sm, dg, w1, b1, w2, b2, eps, out):
    deg = jnp.maximum(dg[...], 1.0)
    mean = sm[...] / deg
    rst = (1.0 + eps[0, 0]) * nf[...] + mean
    h = jnp.maximum(
        jnp.dot(rst, w1[...], preferred_element_type=jnp.float32) + b1[...], 0.0)
    out[...] = jnp.dot(h, w2[...], preferred_element_type=jnp.float32) + b2[...]


def _tc_apply(nf, sm, deg, W1, b1, W2, b2, eps):
    return pl.pallas_call(
        _tc_body,
        grid=(N // _BLK,),
        in_specs=[
            pl.BlockSpec((_BLK, D), lambda i: (i, 0)),
            pl.BlockSpec((_BLK, D), lambda i: (i, 0)),
            pl.BlockSpec((_BLK, 1), lambda i: (i, 0)),
            pl.BlockSpec((D, D), lambda i: (0, 0)),
            pl.BlockSpec((1, D), lambda i: (0, 0)),
            pl.BlockSpec((D, D), lambda i: (0, 0)),
            pl.BlockSpec((1, D), lambda i: (0, 0)),
            pl.BlockSpec((1, 1), lambda i: (0, 0)),
        ],
        out_specs=pl.BlockSpec((_BLK, D), lambda i: (i, 0)),
        out_shape=jax.ShapeDtypeStruct((N, D), jnp.float32),
    )(nf, sm, deg, W1, b1, W2, b2, eps)


def kernel(node_feat, coord_feat, edge_feat, edge_index, W1, b1, W2, b2, eps):
    src = edge_index[0].astype(jnp.int32)
    dst = edge_index[1].astype(jnp.int32)
    pad = E_PAD - E
    src_p = jnp.concatenate([src, jnp.zeros((pad,), jnp.int32)])
    dst_p = jnp.concatenate([dst, jnp.full((pad,), N, jnp.int32)])
    src2 = jnp.stack([src_p, src_p + N]).reshape(NC, NS, TCH, CH)
    dst_u = dst_p.reshape(NS, TCH, CH)
    dst_c = dst_p.reshape(NCHUNK, CH)
    nfh = node_feat.reshape(N, 2, H).transpose(1, 0, 2).reshape(2 * N, H)

    out_sum = _sc_halves(nfh, src2, dst_u)
    deg2 = _sc_degree(dst_c)
    summed = jnp.concatenate([out_sum[0, :N], out_sum[1, :N]], axis=1)
    deg = (deg2[0, :N, 0] + deg2[1, :N, 0])[:, None]

    hx = _tc_apply(node_feat, summed, deg, W1, jnp.reshape(b1, (1, D)),
                   W2, jnp.reshape(b2, (1, D)),
                   jnp.reshape(eps, (1, 1)).astype(jnp.float32))
    return (hx, coord_feat, edge_feat)
